# baseline (device time: 197758 ns/iter reference)
import jax
import jax.numpy as jnp
from jax import lax
from jax.experimental import pallas as pl
from jax.experimental.pallas import tpu as pltpu

N = 8
SQ = 1024
HP = 8
DH = 128
HD = HP * DH
KVW = 1152
KV0 = 1024
KV1 = 128
CH = SQ // N
SCALE = 0.08838834764831843

HH = HP // 2

_RELAYS = {
    1: [("K", 2, 0), ("V", 2, 0), ("K", 2, 1), ("V", 2, 1)],
    3: [("K", 7, 0), ("V", 7, 0), ("K", 7, 1), ("V", 7, 1),
        ("K", 6, 0), ("K", 6, 1)],
    4: [("K", 5, 0), ("V", 5, 0), ("K", 5, 1), ("V", 5, 1),
        ("V", 6, 0), ("V", 6, 1)],
}
_LINK_PLANS = [
    [(t, d, hf, (1, i)) for i, (t, d, hf) in enumerate(_RELAYS[1])]
    + [("K", 1, 0, None), ("V", 1, 0, None),
       ("K", 1, 1, None), ("V", 1, 1, None)],
    [(t, d, hf, (3, i)) for i, (t, d, hf) in enumerate(_RELAYS[3])]
    + [("K", 3, 0, None), ("V", 3, 0, None),
       ("K", 3, 1, None), ("V", 3, 1, None)],
    [(t, d, hf, (4, i)) for i, (t, d, hf) in enumerate(_RELAYS[4])]
    + [("K", 4, 0, None), ("V", 4, 0, None),
       ("K", 4, 1, None), ("V", 4, 1, None)],
]


def _round_robin(plans):
    out = []
    r = 0
    while any(r < len(p) for p in plans):
        for li, p in enumerate(plans):
            if r < len(p):
                out.append((li, p[r]))
        r += 1
    return out


def _body(x_ref, wq_ref, k_hbm, v_hbm, wo_ref, out_ref,
          q16, kbuf, vbuf, ctx16, acc16, red, red16,
          arbuf, agbuf, relay_buf, stage16, stageF,
          kv_send, kv_recv, loc_sem,
          relay_send, relay_recv,
          ar_send, ar_recv, ag_send, ag_recv):
    my = lax.axis_index("i")

    r0 = pl.ds(0, KV0)
    r1 = pl.ds(KV0, KV1)

    def _dst(tensor, half, rows):
        b = kbuf if tensor == "K" else vbuf
        return b.at[pl.ds(half * HH, HH), rows, :]

    def _rslot(tensor, half, src1):
        return (0 if tensor == "K" else 1) + 2 * half + (4 if src1 else 0)

    def _pull_heads(tensor, head0, n_heads, n_rows, dst):
        hbm = k_hbm if tensor == "K" else v_hbm
        cps = []
        for h in range(n_heads):
            cp = pltpu.make_async_copy(
                hbm.at[0, pl.ds(0, n_rows), head0 + h, :],
                dst.at[h, pl.ds(0, n_rows), :],
                loc_sem.at[h])
            cp.start()
            cps.append(cp)
        for cp in cps:
            cp.wait()

    def _stage_send(n_rows, payloads):
        prev = [None] * 6
        counters = [0, 0, 0]
        rows = pl.ds(0, n_rows)
        src1 = n_rows == KV1
        for li, (tensor, dest, half, relay) in payloads:
            slot = 2 * li + (counters[li] % 2)
            counters[li] += 1
            if prev[slot] is not None:
                prev[slot].wait_send()
            _pull_heads(tensor, dest * HP + half * HH, HH, n_rows, stageF)
            stage16[slot, :, rows, :] = \
                stageF[pl.ds(0, HH), rows, :].astype(jnp.bfloat16)
            if relay is None:
                dst = _dst(tensor, half, r1 if src1 else r0)
                rsem = kv_recv.at[_rslot(tensor, half, src1)]
            else:
                rpos, ridx = relay
                dst = relay_buf.at[ridx]
                rsem = relay_recv.at[ridx]
                dest = rpos
            rd = pltpu.make_async_remote_copy(
                src_ref=stage16.at[slot, :, rows, :],
                dst_ref=dst,
                send_sem=kv_send.at[slot],
                recv_sem=rsem,
                device_id=(dest,),
                device_id_type=pl.DeviceIdType.MESH,
            )
            rd.start()
            prev[slot] = rd
        return prev

    @pl.when(my == 0)
    def _():
        prev = _stage_send(KV0, _round_robin(_LINK_PLANS))
        for tensor in ("K", "V"):
            _pull_heads(tensor, 0, HP, KV0, stageF)
            b = kbuf if tensor == "K" else vbuf
            b[:, r0, :] = stageF[...].astype(jnp.bfloat16)
        for d in prev:
            if d is not None:
                d.wait_send()

    @pl.when(my == 1)
    def _():
        flat = []
        for j in (0, 2, 3, 4, 5, 6, 7):
            flat += [("K", j, 0, None), ("V", j, 0, None),
                     ("K", j, 1, None), ("V", j, 1, None)]
        plans1 = [flat[0:10], flat[10:20], flat[20:28]]
        prev = _stage_send(KV1, _round_robin(plans1))
        for tensor in ("K", "V"):
            _pull_heads(tensor, HP, HP, KV1, stageF)
            b = kbuf if tensor == "K" else vbuf
            b[:, r1, :] = stageF[:, pl.ds(0, KV1), :].astype(jnp.bfloat16)
        for d in prev:
            if d is not None:
                d.wait_send()

    q = jnp.dot(x_ref[...].astype(jnp.bfloat16),
                wq_ref[...].astype(jnp.bfloat16),
                preferred_element_type=jnp.float32)
    q16[...] = (q * SCALE).astype(jnp.bfloat16)

    for rpos, plist in _RELAYS.items():
        @pl.when(my == rpos)
        def _(plist=plist):
            for idx, (tensor, dest, half) in enumerate(plist):
                pltpu.make_async_remote_copy(
                    src_ref=relay_buf.at[idx], dst_ref=relay_buf.at[idx],
                    send_sem=kv_send.at[0], recv_sem=relay_recv.at[idx],
                    device_id=(0,), device_id_type=pl.DeviceIdType.MESH,
                ).wait_recv()
                pltpu.make_async_remote_copy(
                    src_ref=relay_buf.at[idx],
                    dst_ref=_dst(tensor, half, r0),
                    send_sem=relay_send.at[idx],
                    recv_sem=kv_recv.at[_rslot(tensor, half, False)],
                    device_id=(dest,),
                    device_id_type=pl.DeviceIdType.MESH,
                ).start()

    def _kv_wait(tensor, half, rows, src1):
        slot = _rslot(tensor, half, src1)
        b = kbuf if tensor == "K" else vbuf
        pltpu.make_async_remote_copy(
            src_ref=b.at[pl.ds(half * HH, HH), rows, :],
            dst_ref=b.at[pl.ds(half * HH, HH), rows, :],
            send_sem=kv_send.at[0], recv_sem=kv_recv.at[slot],
            device_id=(0,), device_id_type=pl.DeviceIdType.MESH,
        ).wait_recv()

    qi = lax.broadcasted_iota(jnp.int32, (SQ, KVW), 0)
    ki = lax.broadcasted_iota(jnp.int32, (SQ, KVW), 1)
    mask = jnp.abs(qi - ki) <= 128

    for half in (0, 1):
        @pl.when(my != 0)
        def _(half=half):
            _kv_wait("K", half, r0, False)
            _kv_wait("V", half, r0, False)

        @pl.when(my != 1)
        def _(half=half):
            _kv_wait("K", half, r1, True)
            _kv_wait("V", half, r1, True)

        for h in range(half * HH, (half + 1) * HH):
            c = pl.ds(h * DH, DH)
            s = lax.dot_general(
                q16[:, c], kbuf[h],
                (((1,), (1,)), ((), ())), preferred_element_type=jnp.float32)
            s = jnp.where(mask, s, -1e9)
            m = jnp.max(s, axis=1, keepdims=True)
            w = jnp.exp(s - m)
            w = w / jnp.sum(w, axis=1, keepdims=True)
            ctx = lax.dot_general(
                w.astype(jnp.bfloat16), vbuf[h],
                (((1,), (0,)), ((), ())), preferred_element_type=jnp.float32)
            ctx16[:, c] = ctx.astype(jnp.bfloat16)

    acc = jnp.dot(ctx16[...], wo_ref[...].astype(jnp.bfloat16),
                  preferred_element_type=jnp.float32)
    acc16[...] = acc.astype(jnp.bfloat16)

    for o in range(1, N):
        dest = lax.rem(my + o, N)
        pltpu.make_async_remote_copy(
            src_ref=acc16.at[pl.ds(dest * CH, CH), :],
            dst_ref=arbuf.at[o - 1],
            send_sem=ar_send.at[o - 1],
            recv_sem=ar_recv.at[o - 1],
            device_id=(dest,),
            device_id_type=pl.DeviceIdType.MESH,
        ).start()
    for o in range(1, N):
        pltpu.make_async_remote_copy(
            src_ref=arbuf.at[o - 1], dst_ref=arbuf.at[o - 1],
            send_sem=ar_send.at[o - 1], recv_sem=ar_recv.at[o - 1],
            device_id=(0,), device_id_type=pl.DeviceIdType.MESH,
        ).wait_recv()

    r = acc16[pl.ds(my * CH, CH), :].astype(jnp.float32)
    for t in range(N - 1):
        r = r + arbuf[t].astype(jnp.float32)
    red[...] = r
    red16[...] = r.astype(jnp.bfloat16)
    out_ref[pl.ds(my * CH, CH), :] = r

    for o in range(1, N):
        dest = lax.rem(my + o, N)
        pltpu.make_async_remote_copy(
            src_ref=red16,
            dst_ref=agbuf.at[o - 1],
            send_sem=ag_send.at[o - 1],
            recv_sem=ag_recv.at[o - 1],
            device_id=(dest,),
            device_id_type=pl.DeviceIdType.MESH,
        ).start()
    for o in range(1, N):
        pltpu.make_async_remote_copy(
            src_ref=agbuf.at[o - 1], dst_ref=agbuf.at[o - 1],
            send_sem=ag_send.at[o - 1], recv_sem=ag_recv.at[o - 1],
            device_id=(0,), device_id_type=pl.DeviceIdType.MESH,
        ).wait_recv()
        src_pos = lax.rem(my - o + N, N)
        out_ref[pl.ds(src_pos * CH, CH), :] = agbuf[o - 1].astype(jnp.float32)

    for rpos, plist in _RELAYS.items():
        @pl.when(my == rpos)
        def _(plist=plist):
            for idx, (tensor, dest, half) in enumerate(plist):
                pltpu.make_async_remote_copy(
                    src_ref=relay_buf.at[idx], dst_ref=_dst(tensor, half, r0),
                    send_sem=relay_send.at[idx],
                    recv_sem=kv_recv.at[_rslot(tensor, half, False)],
                    device_id=(dest,), device_id_type=pl.DeviceIdType.MESH,
                ).wait_send()

    for o in range(1, N):
        pltpu.make_async_remote_copy(
            src_ref=acc16.at[pl.ds(0, CH), :], dst_ref=arbuf.at[o - 1],
            send_sem=ar_send.at[o - 1], recv_sem=ar_recv.at[o - 1],
            device_id=(0,), device_id_type=pl.DeviceIdType.MESH,
        ).wait_send()
        pltpu.make_async_remote_copy(
            src_ref=red16, dst_ref=agbuf.at[o - 1],
            send_sem=ag_send.at[o - 1], recv_sem=ag_recv.at[o - 1],
            device_id=(0,), device_id_type=pl.DeviceIdType.MESH,
        ).wait_send()


def kernel(x, Wq, K_ext, V_ext, Wo):
    out = pl.pallas_call(
        _body,
        out_shape=jax.ShapeDtypeStruct((SQ, 1024), jnp.float32),
        in_specs=[
            pl.BlockSpec(memory_space=pltpu.MemorySpace.VMEM),
            pl.BlockSpec(memory_space=pltpu.MemorySpace.VMEM),
            pl.BlockSpec(memory_space=pltpu.MemorySpace.HBM),
            pl.BlockSpec(memory_space=pltpu.MemorySpace.HBM),
            pl.BlockSpec(memory_space=pltpu.MemorySpace.VMEM),
        ],
        out_specs=pl.BlockSpec(memory_space=pltpu.MemorySpace.VMEM),
        scratch_shapes=[
            pltpu.VMEM((SQ, HD), jnp.bfloat16),
            pltpu.VMEM((HP, KVW, DH), jnp.bfloat16),
            pltpu.VMEM((HP, KVW, DH), jnp.bfloat16),
            pltpu.VMEM((SQ, HD), jnp.bfloat16),
            pltpu.VMEM((SQ, 1024), jnp.bfloat16),
            pltpu.VMEM((CH, 1024), jnp.float32),
            pltpu.VMEM((CH, 1024), jnp.bfloat16),
            pltpu.VMEM((N - 1, CH, 1024), jnp.bfloat16),
            pltpu.VMEM((N - 1, CH, 1024), jnp.bfloat16),
            pltpu.VMEM((6, HH, KV0, DH), jnp.bfloat16),
            pltpu.VMEM((6, HH, KV0, DH), jnp.bfloat16),
            pltpu.VMEM((HP, KV0, DH), jnp.float32),
            pltpu.SemaphoreType.DMA((6,)),
            pltpu.SemaphoreType.DMA((8,)),
            pltpu.SemaphoreType.DMA((HP,)),
            pltpu.SemaphoreType.DMA((6,)),
            pltpu.SemaphoreType.DMA((6,)),
            pltpu.SemaphoreType.DMA((N - 1,)),
            pltpu.SemaphoreType.DMA((N - 1,)),
            pltpu.SemaphoreType.DMA((N - 1,)),
            pltpu.SemaphoreType.DMA((N - 1,)),
        ],
        compiler_params=pltpu.CompilerParams(
            vmem_limit_bytes=100 * 1024 * 1024,
        ),
    )(x.reshape(SQ, 1024), Wq, K_ext, V_ext, Wo)
    return out.reshape(1, SQ, 1024)
